# baseline (speedup 1.0000x reference)
import functools

import jax
import jax.numpy as jnp
from jax import lax
from jax.experimental import pallas as pl
from jax.experimental.pallas import tpu as pltpu
from jax.experimental.pallas import tpu_sc as plsc

_R, _C = 16384, 2048
_SC_ROWS = 1024
_R_TC = _R - _SC_ROWS
_BLK = 1024
_NW = 32
_WROWS = _SC_ROWS // _NW
_CH = 8
_NCH = _WROWS // _CH


_TCH = 8
_TUN = 64


def _tc_vals(x, lab):
    eps = jnp.float32(1e-31)
    y = x * x + 1.0
    s = y * lax.rsqrt(y)
    t = jnp.exp(1.0 - s)
    w = jnp.where(lab == 1, jnp.maximum(1.0 - t, eps), t)
    return jnp.log(w)


def _tc_body(out_ref, lab_ref, sum_ref):
    def step(j, acc):
        x = out_ref[pl.ds(j * _TCH, _TCH), :]
        lab = lab_ref[pl.ds(j * _TCH, _TCH), :]
        return acc + _tc_vals(x, lab)

    acc = jax.lax.fori_loop(
        0, _BLK // _TCH, step, jnp.zeros((_TCH, _C), jnp.float32),
        unroll=_TUN,
    )
    part = jnp.sum(acc)

    @pl.when(pl.program_id(0) == 0)
    def _():
        sum_ref[0, 0] = 0.0

    sum_ref[0, 0] -= part


def _tc_sum(out, label):
    return pl.pallas_call(
        _tc_body,
        grid=(_R_TC // _BLK,),
        in_specs=[
            pl.BlockSpec((_BLK, _C), lambda i: (i, 0)),
            pl.BlockSpec((_BLK, _C), lambda i: (i, 0)),
        ],
        out_specs=pl.BlockSpec(memory_space=pltpu.SMEM),
        out_shape=jax.ShapeDtypeStruct((1, 1), jnp.float32),
    )(out, label)


def _sc_loss_vec(x, l):
    one = jnp.float32(1.0)
    y = x * x + one
    i = lax.bitcast_convert_type(y, jnp.int32)
    r = lax.bitcast_convert_type(jnp.int32(0x5F3759DF) - (i >> 1), jnp.float32)
    half_y = 0.5 * y
    r = r * (1.5 - half_y * r * r)
    r = r * (1.5 - half_y * r * r)
    r = r * (1.5 - half_y * r * r)
    s = y * r
    loss = s - one
    t = jnp.exp(one - s)
    z = jnp.maximum(one - t, jnp.float32(1e-31))
    zi = lax.bitcast_convert_type(z, jnp.int32)
    e = (zi >> 23) - 127
    m = lax.bitcast_convert_type(
        (zi & jnp.int32(0x007FFFFF)) | jnp.int32(0x3F800000), jnp.float32)
    w = (m - one) / (m + one)
    w2 = w * w
    p = one + w2 * (jnp.float32(1 / 3) + w2 * (jnp.float32(0.2)
                                               + w2 * jnp.float32(1 / 7)))
    logz = e.astype(jnp.float32) * jnp.float32(0.6931471805599453) + 2.0 * w * p
    return jnp.where(l == 1, -logz, loss)


def _sc_body(out_hbm, lab_hbm, acc_hbm, xv, lv, accv):
    wid = lax.axis_index("s") * 2 + lax.axis_index("c")
    row0 = _R_TC + wid * _WROWS

    acc = jnp.zeros((16,), jnp.float32)
    for ch in range(_NCH):
        pltpu.sync_copy(out_hbm.at[pl.ds(row0 + ch * _CH, _CH), :], xv)
        pltpu.sync_copy(lab_hbm.at[pl.ds(row0 + ch * _CH, _CH), :], lv)

        def step(i, a):
            for rr in range(_CH):
                x = xv[rr, pl.ds(i * 16, 16)]
                l = lv[rr, pl.ds(i * 16, 16)]
                a = a + _sc_loss_vec(x, l)
            return a

        acc = lax.fori_loop(0, _C // 16, step, acc)

    accv[...] = acc
    pltpu.sync_copy(accv, acc_hbm.at[wid])


_SC_SUM_CACHE = []


def _sc_sum():
    if not _SC_SUM_CACHE:
        _SC_SUM_CACHE.append(functools.partial(
            pl.kernel,
            mesh=plsc.VectorSubcoreMesh(core_axis_name="c", subcore_axis_name="s"),
            out_type=jax.ShapeDtypeStruct((_NW, 16), jnp.float32),
            scratch_types=[
                pltpu.VMEM((_CH, _C), jnp.float32),
                pltpu.VMEM((_CH, _C), jnp.int32),
                pltpu.VMEM((16,), jnp.float32),
            ],
        )(_sc_body))
    return _SC_SUM_CACHE[0]


def kernel(out, label):
    sc_acc = _sc_sum()(out, label)
    tc_total = _tc_sum(out, label)
    return (tc_total[0, 0] + jnp.sum(sc_acc)) * (1.0 / (_R * _C))


# speedup vs baseline: 1.2000x; 1.2000x over previous
import functools

import jax
import jax.numpy as jnp
from jax import lax
from jax.experimental import pallas as pl
from jax.experimental.pallas import tpu as pltpu
from jax.experimental.pallas import tpu_sc as plsc

_R, _C = 16384, 2048
_SC_ROWS = 0
_R_TC = _R - _SC_ROWS
_BLK = 1024
_NW = 32
_WROWS = _SC_ROWS // _NW
_CH = 8
_NCH = _WROWS // _CH


_TCH = 8
_TUN = 128


def _tc_vals(x, lab):
    eps = jnp.float32(1e-31)
    y = x * x + 1.0
    s = y * lax.rsqrt(y)
    t = jnp.exp(1.0 - s)
    w = jnp.where(lab == 1, jnp.maximum(1.0 - t, eps), t)
    return jnp.log(w)


def _tc_body(out_ref, lab_ref, sum_ref):
    def step(j, acc):
        x = out_ref[pl.ds(j * _TCH, _TCH), :]
        lab = lab_ref[pl.ds(j * _TCH, _TCH), :]
        return acc + _tc_vals(x, lab)

    acc = jax.lax.fori_loop(
        0, _BLK // _TCH, step, jnp.zeros((_TCH, _C), jnp.float32),
        unroll=_TUN,
    )
    part = jnp.sum(acc)

    @pl.when(pl.program_id(0) == 0)
    def _():
        sum_ref[0, 0] = 0.0

    sum_ref[0, 0] -= part


def _tc_sum(out, label):
    return pl.pallas_call(
        _tc_body,
        grid=(_R_TC // _BLK,),
        in_specs=[
            pl.BlockSpec((_BLK, _C), lambda i: (i, 0)),
            pl.BlockSpec((_BLK, _C), lambda i: (i, 0)),
        ],
        out_specs=pl.BlockSpec(memory_space=pltpu.SMEM),
        out_shape=jax.ShapeDtypeStruct((1, 1), jnp.float32),
    )(out, label)


def _sc_loss_vec(x, l):
    one = jnp.float32(1.0)
    y = x * x + one
    i = lax.bitcast_convert_type(y, jnp.int32)
    r = lax.bitcast_convert_type(jnp.int32(0x5F3759DF) - (i >> 1), jnp.float32)
    half_y = 0.5 * y
    r = r * (1.5 - half_y * r * r)
    r = r * (1.5 - half_y * r * r)
    r = r * (1.5 - half_y * r * r)
    s = y * r
    loss = s - one
    t = jnp.exp(one - s)
    z = jnp.maximum(one - t, jnp.float32(1e-31))
    zi = lax.bitcast_convert_type(z, jnp.int32)
    e = (zi >> 23) - 127
    m = lax.bitcast_convert_type(
        (zi & jnp.int32(0x007FFFFF)) | jnp.int32(0x3F800000), jnp.float32)
    w = (m - one) / (m + one)
    w2 = w * w
    p = one + w2 * (jnp.float32(1 / 3) + w2 * (jnp.float32(0.2)
                                               + w2 * jnp.float32(1 / 7)))
    logz = e.astype(jnp.float32) * jnp.float32(0.6931471805599453) + 2.0 * w * p
    return jnp.where(l == 1, -logz, loss)


def _sc_body(out_hbm, lab_hbm, acc_hbm, xv, lv, accv):
    wid = lax.axis_index("s") * 2 + lax.axis_index("c")
    row0 = _R_TC + wid * _WROWS

    acc = jnp.zeros((16,), jnp.float32)
    for ch in range(_NCH):
        pltpu.sync_copy(out_hbm.at[pl.ds(row0 + ch * _CH, _CH), :], xv)
        pltpu.sync_copy(lab_hbm.at[pl.ds(row0 + ch * _CH, _CH), :], lv)

        def step(i, a):
            for rr in range(_CH):
                x = xv[rr, pl.ds(i * 16, 16)]
                l = lv[rr, pl.ds(i * 16, 16)]
                a = a + _sc_loss_vec(x, l)
            return a

        acc = lax.fori_loop(0, _C // 16, step, acc)

    accv[...] = acc
    pltpu.sync_copy(accv, acc_hbm.at[wid])


_SC_SUM_CACHE = []


def _sc_sum():
    if not _SC_SUM_CACHE:
        _SC_SUM_CACHE.append(functools.partial(
            pl.kernel,
            mesh=plsc.VectorSubcoreMesh(core_axis_name="c", subcore_axis_name="s"),
            out_type=jax.ShapeDtypeStruct((_NW, 16), jnp.float32),
            scratch_types=[
                pltpu.VMEM((_CH, _C), jnp.float32),
                pltpu.VMEM((_CH, _C), jnp.int32),
                pltpu.VMEM((16,), jnp.float32),
            ],
        )(_sc_body))
    return _SC_SUM_CACHE[0]


def kernel(out, label):
    tc_total = _tc_sum(out, label)
    return tc_total[0, 0] * (1.0 / (_R * _C))
